# baseline (speedup 1.0000x reference)
import functools

import jax
import jax.numpy as jnp
import numpy as np
from jax import lax
from jax.experimental import pallas as pl
from jax.experimental.pallas import tpu as pltpu
from jax.experimental.pallas import tpu_sc as plsc

D_LANES = 16


def _pos_encoding(seq_len, d_model):
    pos = np.arange(seq_len, dtype=np.float32)[:, None]
    i = np.arange(0, d_model, 2, dtype=np.float32)
    angle = (pos / np.power(np.float32(10000.0), i / np.float32(d_model))
             ).astype(np.float32)
    pe = np.zeros((seq_len, d_model), dtype=np.float32)
    pe[:, 0::2] = np.sin(angle)
    pe[:, 1::2] = np.cos(angle)
    return jnp.asarray(pe.reshape(-1))


def _make_sc_kernel(B, S, D, s_per_w, chunk, nbuf=4):
    n_chunks = s_per_w // chunk
    n_tasks = n_chunks * B
    vecs_per_row = D // D_LANES
    mesh = plsc.VectorSubcoreMesh(core_axis_name="c", subcore_axis_name="s")
    info = plsc.get_sparse_core_info()
    nc = info.num_cores

    assert nbuf == B, "slot = batch index requires nbuf == B"
    n_pairs = (n_chunks - 2) // 2

    @functools.partial(
        pl.kernel,
        mesh=mesh,
        out_type=jax.ShapeDtypeStruct((B * S, D), jnp.float32),
        scratch_types=[
            pltpu.VMEM((B, s_per_w), jnp.int32),
            pltpu.VMEM((2, chunk * D), jnp.float32),
            pltpu.VMEM((nbuf, chunk, D), jnp.float32),
            pltpu.SemaphoreType.DMA,
            pltpu.SemaphoreType.DMA,
        ] + [pltpu.SemaphoreType.DMA] * (2 * nbuf),
    )
    def k(x_hbm, table_hbm, pos_hbm, out_hbm, idx_all, pos_v, rows, p0, p1, *sems):
        wid = lax.axis_index("s") * nc + lax.axis_index("c")
        w_s0 = wid * s_per_w
        psem = [p0, p1]
        gsem = list(sems[:nbuf])
        ssem = list(sems[nbuf:])

        for b in range(B):
            pltpu.sync_copy(x_hbm.at[b, pl.ds(w_s0, s_per_w)], idx_all.at[b])

        def gather_issue(kc, b):
            pltpu.async_copy(
                table_hbm.at[idx_all.at[b, pl.ds(kc * chunk, chunk)]],
                rows.at[b], gsem[b])

        def gather_wait(b):
            pltpu.make_async_copy(
                table_hbm.at[pl.ds(0, chunk)], rows.at[b], gsem[b]).wait()

        def store_issue(kc, b):
            pltpu.async_copy(
                rows.at[b],
                out_hbm.at[pl.ds(b * S + w_s0 + kc * chunk, chunk)], ssem[b])

        def store_wait(b):
            pltpu.make_async_copy(
                rows.at[b], out_hbm.at[pl.ds(0, chunk)], ssem[b]).wait()

        def pos_issue(kc, par):
            pltpu.async_copy(
                pos_hbm.at[pl.ds((w_s0 + kc * chunk) * D, chunk * D)],
                pos_v.at[par], psem[par])

        def pos_wait(par):
            pltpu.make_async_copy(
                pos_hbm.at[pl.ds(0, chunk * D)], pos_v.at[par], psem[par]).wait()

        def process(kc, b, par, skip_swait=False, gnext=None):
            if not skip_swait:
                store_wait((b + 2) % nbuf)
            if gnext is not None:
                gather_issue(*gnext)
            gather_wait(b)
            pv = pos_v.at[par]

            def body(r, _):
                base = r * D
                for j2 in range(vecs_per_row):
                    off = j2 * D_LANES
                    plsc.addupdate(rows.at[b, r, pl.ds(off, D_LANES)],
                                   pv[pl.ds(base + off, D_LANES)])
                return 0

            lax.fori_loop(0, chunk, body, 0)
            store_issue(kc, b)

        pos_issue(0, 0)
        gather_issue(0, 0)
        gather_issue(0, 1)
        pos_wait(0)
        pos_issue(1, 1)
        process(0, 0, 0, skip_swait=True, gnext=(0, 2))
        process(0, 1, 0, skip_swait=True, gnext=(0, 3))
        process(0, 2, 0, gnext=(1, 0))
        process(0, 3, 0, gnext=(1, 1))

        def pair_body(p, _):
            kc = 1 + 2 * p
            pos_wait(1)
            pos_issue(kc + 1, 0)
            process(kc, 0, 1, gnext=(kc, 2))
            process(kc, 1, 1, gnext=(kc, 3))
            process(kc, 2, 1, gnext=(kc + 1, 0))
            process(kc, 3, 1, gnext=(kc + 1, 1))
            pos_wait(0)
            pos_issue(kc + 2, 1)
            process(kc + 1, 0, 0, gnext=(kc + 1, 2))
            process(kc + 1, 1, 0, gnext=(kc + 1, 3))
            process(kc + 1, 2, 0, gnext=(kc + 2, 0))
            process(kc + 1, 3, 0, gnext=(kc + 2, 1))
            return 0

        lax.fori_loop(0, n_pairs, pair_body, 0)

        last = n_chunks - 1
        pos_wait(last & 1)
        process(last, 0, last & 1, gnext=(last, 2))
        process(last, 1, last & 1, gnext=(last, 3))
        process(last, 2, last & 1)
        process(last, 3, last & 1)
        store_wait(2)
        store_wait(3)

    return k


def kernel(x, token_table):
    B, S = x.shape
    V, D = token_table.shape
    n_workers = 32
    s_per_w = S // n_workers
    pos = _pos_encoding(S, D)
    k = _make_sc_kernel(B, S, D, s_per_w, chunk=16)
    out = k(x, token_table, pos)
    return out.reshape(B, S, D)


# speedup vs baseline: 1.4463x; 1.4463x over previous
import functools

import jax
import jax.numpy as jnp
import numpy as np
from jax import lax
from jax.experimental import pallas as pl
from jax.experimental.pallas import tpu as pltpu
from jax.experimental.pallas import tpu_sc as plsc

D_LANES = 16


def _pos_encoding(seq_len, d_model):
    pos = np.arange(seq_len, dtype=np.float32)[:, None]
    i = np.arange(0, d_model, 2, dtype=np.float32)
    angle = (pos / np.power(np.float32(10000.0), i / np.float32(d_model))
             ).astype(np.float32)
    pe = np.zeros((seq_len, d_model), dtype=np.float32)
    pe[:, 0::2] = np.sin(angle)
    pe[:, 1::2] = np.cos(angle)
    return jnp.asarray(pe)


def _make_sc_kernel(B, S, D, s_per_w, chunk, nbuf=4):
    n_chunks = s_per_w // chunk
    n_tasks = n_chunks * B
    vecs_per_row = D // D_LANES
    mesh = plsc.VectorSubcoreMesh(core_axis_name="c", subcore_axis_name="s")
    info = plsc.get_sparse_core_info()
    nc = info.num_cores

    assert nbuf == B, "slot = batch index requires nbuf == B"
    n_pairs = (n_chunks - 2) // 2

    @functools.partial(
        pl.kernel,
        mesh=mesh,
        out_type=jax.ShapeDtypeStruct((B * S, D), jnp.float32),
        scratch_types=[
            pltpu.VMEM((B, s_per_w), jnp.int32),
            pltpu.VMEM((2, chunk, D), jnp.float32),
            pltpu.VMEM((nbuf, chunk, D), jnp.float32),
            pltpu.SemaphoreType.DMA,
            pltpu.SemaphoreType.DMA,
        ] + [pltpu.SemaphoreType.DMA] * (2 * nbuf),
    )
    def k(x_hbm, table_hbm, pos_hbm, out_hbm, idx_all, pos_v, rows, p0, p1, *sems):
        wid = lax.axis_index("s") * nc + lax.axis_index("c")
        w_s0 = wid * s_per_w
        psem = [p0, p1]
        gsem = list(sems[:nbuf])
        ssem = list(sems[nbuf:])

        for b in range(B):
            pltpu.sync_copy(x_hbm.at[b, pl.ds(w_s0, s_per_w)], idx_all.at[b])

        def gather_issue(kc, b):
            pltpu.async_copy(
                table_hbm.at[idx_all.at[b, pl.ds(kc * chunk, chunk)]],
                rows.at[b], gsem[b])

        def gather_wait(b):
            pltpu.make_async_copy(
                table_hbm.at[pl.ds(0, chunk)], rows.at[b], gsem[b]).wait()

        def store_issue(kc, b):
            pltpu.async_copy(
                rows.at[b],
                out_hbm.at[pl.ds(b * S + w_s0 + kc * chunk, chunk)], ssem[b])

        def store_wait(b):
            pltpu.make_async_copy(
                rows.at[b], out_hbm.at[pl.ds(0, chunk)], ssem[b]).wait()

        def pos_issue(kc, par):
            pltpu.async_copy(
                pos_hbm.at[pl.ds(w_s0 + kc * chunk, chunk)],
                pos_v.at[par], psem[par])

        def pos_wait(par):
            pltpu.make_async_copy(
                pos_hbm.at[pl.ds(0, chunk)], pos_v.at[par], psem[par]).wait()

        def process(kc, b, par, skip_swait=False, gnext=None):
            if not skip_swait:
                store_wait((b + 2) % nbuf)
            if gnext is not None:
                gather_issue(*gnext)
            gather_wait(b)
            pv = pos_v.at[par]

            def body(r, _):
                for j2 in range(vecs_per_row):
                    sl = pl.ds(j2 * D_LANES, D_LANES)
                    plsc.addupdate(rows.at[b, r, sl], pv[r, sl])
                return 0

            lax.fori_loop(0, chunk, body, 0)
            store_issue(kc, b)

        pos_issue(0, 0)
        gather_issue(0, 0)
        gather_issue(0, 1)
        pos_wait(0)
        pos_issue(1, 1)
        process(0, 0, 0, skip_swait=True, gnext=(0, 2))
        process(0, 1, 0, skip_swait=True, gnext=(0, 3))
        process(0, 2, 0, gnext=(1, 0))
        process(0, 3, 0, gnext=(1, 1))

        def pair_body(p, _):
            kc = 1 + 2 * p
            pos_wait(1)
            pos_issue(kc + 1, 0)
            process(kc, 0, 1, gnext=(kc, 2))
            process(kc, 1, 1, gnext=(kc, 3))
            process(kc, 2, 1, gnext=(kc + 1, 0))
            process(kc, 3, 1, gnext=(kc + 1, 1))
            pos_wait(0)
            pos_issue(kc + 2, 1)
            process(kc + 1, 0, 0, gnext=(kc + 1, 2))
            process(kc + 1, 1, 0, gnext=(kc + 1, 3))
            process(kc + 1, 2, 0, gnext=(kc + 2, 0))
            process(kc + 1, 3, 0, gnext=(kc + 2, 1))
            return 0

        lax.fori_loop(0, n_pairs, pair_body, 0)

        last = n_chunks - 1
        pos_wait(last & 1)
        process(last, 0, last & 1, gnext=(last, 2))
        process(last, 1, last & 1, gnext=(last, 3))
        process(last, 2, last & 1)
        process(last, 3, last & 1)
        store_wait(2)
        store_wait(3)

    return k


def kernel(x, token_table):
    B, S = x.shape
    V, D = token_table.shape
    n_workers = 32
    s_per_w = S // n_workers
    pos = _pos_encoding(S, D)
    k = _make_sc_kernel(B, S, D, s_per_w, chunk=16)
    out = k(x, token_table, pos)
    return out.reshape(B, S, D)
